# baseline (speedup 1.0000x reference)
import functools

import jax
import jax.numpy as jnp
from jax import lax
from jax.experimental import pallas as pl
from jax.experimental.pallas import tpu as pltpu
from jax.experimental.pallas import tpu_sc as plsc

NUM_EMB = 100000
D_BYTES = 128
B_TOTAL = 4096 * 26
NC, NS, L = 2, 16, 16
NW = NC * NS
ROWS_PER_W = B_TOTAL // NW
CHUNK = 128
CHUNKS_PER_W = ROWS_PER_W // CHUNK
TC_BLK = 4000


def _pack_body(in_ref, out_ref):
    out_ref[...] = pltpu.bitcast(in_ref[...], jnp.int32)


def _pack_words(qweight):
    return pl.pallas_call(
        _pack_body,
        grid=(NUM_EMB // TC_BLK,),
        in_specs=[pl.BlockSpec((TC_BLK, D_BYTES), lambda i: (i, 0))],
        out_specs=pl.BlockSpec((TC_BLK // 4, D_BYTES), lambda i: (i, 0)),
        out_shape=jax.ShapeDtypeStruct((NUM_EMB // 4, D_BYTES), jnp.int32),
    )(qweight)


def _body(qw_hbm, idx_hbm, scale_hbm, bias_hbm, out_hbm,
          idx_v, idxg0_v, idxg1_v, sh0_v, sh1_v,
          rows0_v, rows1_v, out0_v, out1_v, sv_v, bv_v,
          gsem0, gsem1, osem0, osem1):
    wid = lax.axis_index("s") * NC + lax.axis_index("c")

    pltpu.sync_copy(scale_hbm, sv_v)
    pltpu.sync_copy(bias_hbm, bv_v)
    pltpu.sync_copy(idx_hbm.at[pl.ds(wid * CHUNKS_PER_W, CHUNKS_PER_W)], idx_v)
    scale = sv_v[...]
    bias = bv_v[...]

    rows = (rows0_v, rows1_v)
    outs = (out0_v, out1_v)
    idxgs = (idxg0_v, idxg1_v)
    shs = (sh0_v, sh1_v)
    gsems = (gsem0, gsem1)
    osems = (osem0, osem1)

    def start_gather(c):
        b = c & 1
        for i in range(CHUNK // L):
            sl = pl.ds(L * i, L)
            v = idx_v[c, sl]
            idxgs[b][sl] = lax.shift_right_logical(v, 2)
            shs[b][sl] = 24 - lax.shift_left(v & 3, 3)
        return pltpu.async_copy(qw_hbm.at[idxgs[b]], rows[b], gsems[b])

    def dequant(c):
        b = c & 1
        rows_b, out_b, sh_b = rows[b], outs[b], shs[b]

        def do_row(r, _):
            shvec = plsc.load_gather(sh_b, [jnp.full((L,), r, jnp.int32)])
            rowoff = r * D_BYTES
            for w in range(D_BYTES // L):
                x = rows_b[r, pl.ds(L * w, L)]
                t = lax.shift_right_arithmetic(
                    lax.shift_left(x, shvec), 24)
                f = t.astype(jnp.float32) * scale + bias
                out_b[pl.ds(rowoff + L * w, L)] = f
            return 0

        lax.fori_loop(0, CHUNK, do_row, 0)

    def start_writeback(c):
        b = c & 1
        dst = out_hbm.at[pl.ds((wid * CHUNKS_PER_W + c) * CHUNK * D_BYTES,
                               CHUNK * D_BYTES)]
        return pltpu.async_copy(outs[b], dst, osems[b])

    gh = [None] * CHUNKS_PER_W
    oh = [None] * CHUNKS_PER_W
    gh[0] = start_gather(0)
    for c in range(CHUNKS_PER_W):
        if c + 1 < CHUNKS_PER_W:
            gh[c + 1] = start_gather(c + 1)
        gh[c].wait()
        if c >= 2:
            oh[c - 2].wait()
        dequant(c)
        oh[c] = start_writeback(c)
    oh[CHUNKS_PER_W - 2].wait()
    oh[CHUNKS_PER_W - 1].wait()


@jax.jit
def _run(qwords, idx2d, scale_vec, bias_vec):
    mesh = plsc.VectorSubcoreMesh(
        core_axis_name="c", subcore_axis_name="s",
        num_cores=NC, num_subcores=NS)
    return pl.kernel(
        _body,
        out_type=jax.ShapeDtypeStruct((B_TOTAL * D_BYTES,), jnp.float32),
        mesh=mesh,
        scratch_types=[
            pltpu.VMEM((CHUNKS_PER_W, CHUNK), jnp.int32),
            pltpu.VMEM((CHUNK,), jnp.int32),
            pltpu.VMEM((CHUNK,), jnp.int32),
            pltpu.VMEM((CHUNK,), jnp.int32),
            pltpu.VMEM((CHUNK,), jnp.int32),
            pltpu.VMEM((CHUNK, D_BYTES), jnp.int32),
            pltpu.VMEM((CHUNK, D_BYTES), jnp.int32),
            pltpu.VMEM((CHUNK * D_BYTES,), jnp.float32),
            pltpu.VMEM((CHUNK * D_BYTES,), jnp.float32),
            pltpu.VMEM((L,), jnp.float32),
            pltpu.VMEM((L,), jnp.float32),
            pltpu.SemaphoreType.DMA,
            pltpu.SemaphoreType.DMA,
            pltpu.SemaphoreType.DMA,
            pltpu.SemaphoreType.DMA,
        ],
        compiler_params=pltpu.CompilerParams(
            needs_layout_passes=False, use_tc_tiling_on_sc=False),
    )(qwords, idx2d, scale_vec, bias_vec)


def kernel(input, qweight, scale, zero_point):
    nb, nf = input.shape
    qwords = _pack_words(qweight)
    idxT = jnp.swapaxes(input.astype(jnp.int32), 0, 1)
    idx2d = idxT.reshape(B_TOTAL // CHUNK, CHUNK)
    scale_f = scale.astype(jnp.float32)
    bias_f = -zero_point.astype(jnp.float32) * scale_f
    scale_vec = jnp.full((L,), scale_f, jnp.float32)
    bias_vec = jnp.full((L,), bias_f, jnp.float32)
    flat = _run(qwords, idx2d, scale_vec, bias_vec)
    return jnp.transpose(flat.reshape(nf, nb, D_BYTES), (1, 0, 2))


# speedup vs baseline: 3.0855x; 3.0855x over previous
import functools

import jax
import jax.numpy as jnp
from jax import lax
from jax.experimental import pallas as pl
from jax.experimental.pallas import tpu as pltpu
from jax.experimental.pallas import tpu_sc as plsc

NUM_EMB = 100000
D_BYTES = 128
B_TOTAL = 4096 * 26
NC, NS, L = 2, 16, 16
NW = NC * NS
ROWS_PER_W = B_TOTAL // NW
CHUNK = 128
CHUNKS_PER_W = ROWS_PER_W // CHUNK
TC_BLK = 4000


def _pack_body(in_ref, out_ref):
    out_ref[...] = pltpu.bitcast(in_ref[...], jnp.int32)


def _pack_words(qweight):
    return pl.pallas_call(
        _pack_body,
        grid=(NUM_EMB // TC_BLK,),
        in_specs=[pl.BlockSpec((TC_BLK, D_BYTES), lambda i: (i, 0))],
        out_specs=pl.BlockSpec((TC_BLK // 4, D_BYTES), lambda i: (i, 0)),
        out_shape=jax.ShapeDtypeStruct((NUM_EMB // 4, D_BYTES), jnp.int32),
    )(qweight)


def _body(qw_hbm, idx_hbm, scale_hbm, bias_hbm, out_hbm,
          idx_v, idxg0_v, idxg1_v, sh0_v, sh1_v,
          rows0_v, rows1_v, out0_v, out1_v, sv_v, bv_v,
          gsem0, gsem1, osem0, osem1):
    wid = lax.axis_index("s") * NC + lax.axis_index("c")

    pltpu.sync_copy(scale_hbm, sv_v)
    pltpu.sync_copy(bias_hbm, bv_v)
    pltpu.sync_copy(idx_hbm.at[pl.ds(wid * CHUNKS_PER_W, CHUNKS_PER_W)], idx_v)
    scale = sv_v[...]
    bias = bv_v[...]

    rows = (rows0_v, rows1_v)
    outs = (out0_v, out1_v)
    idxgs = (idxg0_v, idxg1_v)
    shs = (sh0_v, sh1_v)
    gsems = (gsem0, gsem1)
    osems = (osem0, osem1)

    def start_gather(c, b):
        for i in range(CHUNK // L):
            sl = pl.ds(L * i, L)
            v = idx_v[c, sl]
            idxgs[b][sl] = lax.shift_right_logical(v, 2)
            shs[b][sl] = lax.shift_left(1, 24 - lax.shift_left(v & 3, 3))
        return pltpu.async_copy(qw_hbm.at[idxgs[b]], rows[b], gsems[b])

    def wait_gather(b):
        pltpu.make_async_copy(qw_hbm.at[idxgs[b]], rows[b], gsems[b]).wait()

    def dequant(b):
        rows_b, out_b, sh_b = rows[b], outs[b], shs[b]

        @plsc.parallel_loop(0, CHUNK, 1, unroll=2)
        def _row(rr):
            mvec = plsc.load_gather(sh_b, [jnp.full((L,), rr, jnp.int32)])
            rowoff = rr * D_BYTES
            for w in range(D_BYTES // L):
                x = rows_b[rr, pl.ds(L * w, L)]
                t = lax.shift_right_arithmetic(x * mvec, 24)
                f = t.astype(jnp.float32) * scale + bias
                out_b[pl.ds(rowoff + L * w, L)] = f

    def out_slice(c):
        return out_hbm.at[pl.ds((wid * CHUNKS_PER_W + c) * CHUNK * D_BYTES,
                                CHUNK * D_BYTES)]

    def start_writeback(c, b):
        return pltpu.async_copy(outs[b], out_slice(c), osems[b])

    def wait_writeback(b):
        pltpu.make_async_copy(outs[b], out_slice(0), osems[b]).wait()

    start_gather(0, 0)

    def step(t, _):
        c0 = 2 * t
        start_gather(c0 + 1, 1)
        wait_gather(0)

        @pl.when(t > 0)
        def _():
            wait_writeback(0)
        dequant(0)
        start_writeback(c0, 0)

        @pl.when(t + 1 < CHUNKS_PER_W // 2)
        def _():
            start_gather(c0 + 2, 0)
        wait_gather(1)

        @pl.when(t > 0)
        def _():
            wait_writeback(1)
        dequant(1)
        start_writeback(c0 + 1, 1)
        return 0

    lax.fori_loop(0, CHUNKS_PER_W // 2, step, 0)
    wait_writeback(0)
    wait_writeback(1)


@jax.jit
def _run(qwords, idx2d, scale_vec, bias_vec):
    mesh = plsc.VectorSubcoreMesh(
        core_axis_name="c", subcore_axis_name="s",
        num_cores=NC, num_subcores=NS)
    return pl.kernel(
        _body,
        out_type=jax.ShapeDtypeStruct((B_TOTAL * D_BYTES,), jnp.float32),
        mesh=mesh,
        scratch_types=[
            pltpu.VMEM((CHUNKS_PER_W, CHUNK), jnp.int32),
            pltpu.VMEM((CHUNK,), jnp.int32),
            pltpu.VMEM((CHUNK,), jnp.int32),
            pltpu.VMEM((CHUNK,), jnp.int32),
            pltpu.VMEM((CHUNK,), jnp.int32),
            pltpu.VMEM((CHUNK, D_BYTES), jnp.int32),
            pltpu.VMEM((CHUNK, D_BYTES), jnp.int32),
            pltpu.VMEM((CHUNK * D_BYTES,), jnp.float32),
            pltpu.VMEM((CHUNK * D_BYTES,), jnp.float32),
            pltpu.VMEM((L,), jnp.float32),
            pltpu.VMEM((L,), jnp.float32),
            pltpu.SemaphoreType.DMA,
            pltpu.SemaphoreType.DMA,
            pltpu.SemaphoreType.DMA,
            pltpu.SemaphoreType.DMA,
        ],
        compiler_params=pltpu.CompilerParams(
            needs_layout_passes=False, use_tc_tiling_on_sc=False),
    )(qwords, idx2d, scale_vec, bias_vec)


def kernel(input, qweight, scale, zero_point):
    nb, nf = input.shape
    qwords = _pack_words(qweight)
    idxT = jnp.swapaxes(input.astype(jnp.int32), 0, 1)
    idx2d = idxT.reshape(B_TOTAL // CHUNK, CHUNK)
    scale_f = scale.astype(jnp.float32)
    bias_f = -zero_point.astype(jnp.float32) * scale_f
    scale_vec = jnp.full((L,), scale_f, jnp.float32)
    bias_vec = jnp.full((L,), bias_f, jnp.float32)
    flat = _run(qwords, idx2d, scale_vec, bias_vec)
    return jnp.transpose(flat.reshape(nf, nb, D_BYTES), (1, 0, 2))


# speedup vs baseline: 3.0916x; 1.0020x over previous
import functools

import jax
import jax.numpy as jnp
from jax import lax
from jax.experimental import pallas as pl
from jax.experimental.pallas import tpu as pltpu
from jax.experimental.pallas import tpu_sc as plsc

NUM_EMB = 100000
D_BYTES = 128
B_TOTAL = 4096 * 26
NC, NS, L = 2, 16, 16
NW = NC * NS
ROWS_PER_W = B_TOTAL // NW
CHUNK = 128
CHUNKS_PER_W = ROWS_PER_W // CHUNK
TC_BLK = 4000


def _pack_body(in_ref, out_ref):
    out_ref[...] = pltpu.bitcast(in_ref[...], jnp.int32)


def _pack_words(qweight):
    return pl.pallas_call(
        _pack_body,
        grid=(NUM_EMB // TC_BLK,),
        in_specs=[pl.BlockSpec((TC_BLK, D_BYTES), lambda i: (i, 0))],
        out_specs=pl.BlockSpec((TC_BLK // 4, D_BYTES), lambda i: (i, 0)),
        out_shape=jax.ShapeDtypeStruct((NUM_EMB // 4, D_BYTES), jnp.int32),
    )(qweight)


def _body(qw_hbm, idx_hbm, scale_hbm, bias_hbm, out_hbm,
          idx_v, idxg0_v, idxg1_v, sh0_v, sh1_v,
          rows0_v, rows1_v, out0_v, out1_v, sv_v, bv_v,
          gsem0, gsem1, osem0, osem1):
    wid = lax.axis_index("s") * NC + lax.axis_index("c")

    pltpu.sync_copy(scale_hbm, sv_v)
    pltpu.sync_copy(bias_hbm, bv_v)
    pltpu.sync_copy(idx_hbm.at[pl.ds(wid * CHUNKS_PER_W, CHUNKS_PER_W)], idx_v)
    scale = sv_v[...]
    bias = bv_v[...]

    rows = (rows0_v, rows1_v)
    outs = (out0_v, out1_v)
    idxgs = (idxg0_v, idxg1_v)
    shs = (sh0_v, sh1_v)
    gsems = (gsem0, gsem1)
    osems = (osem0, osem1)

    def start_gather(c, b):
        for i in range(CHUNK // L):
            sl = pl.ds(L * i, L)
            v = idx_v[c, sl]
            idxgs[b][sl] = lax.shift_right_logical(v, 2)
            shs[b][sl] = lax.shift_left(1, 24 - lax.shift_left(v & 3, 3))
        return pltpu.async_copy(qw_hbm.at[idxgs[b]], rows[b], gsems[b])

    def wait_gather(b):
        pltpu.make_async_copy(qw_hbm.at[idxgs[b]], rows[b], gsems[b]).wait()

    def dequant(b):
        rows_b, out_b, sh_b = rows[b], outs[b], shs[b]

        @plsc.parallel_loop(0, CHUNK, 1, unroll=4)
        def _row(rr):
            mvec = plsc.load_gather(sh_b, [jnp.full((L,), rr, jnp.int32)])
            rowoff = rr * D_BYTES
            for w in range(D_BYTES // L):
                x = rows_b[rr, pl.ds(L * w, L)]
                t = lax.shift_right_arithmetic(x * mvec, 24)
                f = t.astype(jnp.float32) * scale + bias
                out_b[pl.ds(rowoff + L * w, L)] = f

    def out_slice(c):
        return out_hbm.at[pl.ds((wid * CHUNKS_PER_W + c) * CHUNK * D_BYTES,
                                CHUNK * D_BYTES)]

    def start_writeback(c, b):
        return pltpu.async_copy(outs[b], out_slice(c), osems[b])

    def wait_writeback(b):
        pltpu.make_async_copy(outs[b], out_slice(0), osems[b]).wait()

    start_gather(0, 0)

    def step(t, _):
        c0 = 2 * t
        start_gather(c0 + 1, 1)
        wait_gather(0)

        @pl.when(t > 0)
        def _():
            wait_writeback(0)
        dequant(0)
        start_writeback(c0, 0)

        @pl.when(t + 1 < CHUNKS_PER_W // 2)
        def _():
            start_gather(c0 + 2, 0)
        wait_gather(1)

        @pl.when(t > 0)
        def _():
            wait_writeback(1)
        dequant(1)
        start_writeback(c0 + 1, 1)
        return 0

    lax.fori_loop(0, CHUNKS_PER_W // 2, step, 0)
    wait_writeback(0)
    wait_writeback(1)


@jax.jit
def _run(qwords, idx2d, scale_vec, bias_vec):
    mesh = plsc.VectorSubcoreMesh(
        core_axis_name="c", subcore_axis_name="s",
        num_cores=NC, num_subcores=NS)
    return pl.kernel(
        _body,
        out_type=jax.ShapeDtypeStruct((B_TOTAL * D_BYTES,), jnp.float32),
        mesh=mesh,
        scratch_types=[
            pltpu.VMEM((CHUNKS_PER_W, CHUNK), jnp.int32),
            pltpu.VMEM((CHUNK,), jnp.int32),
            pltpu.VMEM((CHUNK,), jnp.int32),
            pltpu.VMEM((CHUNK,), jnp.int32),
            pltpu.VMEM((CHUNK,), jnp.int32),
            pltpu.VMEM((CHUNK, D_BYTES), jnp.int32),
            pltpu.VMEM((CHUNK, D_BYTES), jnp.int32),
            pltpu.VMEM((CHUNK * D_BYTES,), jnp.float32),
            pltpu.VMEM((CHUNK * D_BYTES,), jnp.float32),
            pltpu.VMEM((L,), jnp.float32),
            pltpu.VMEM((L,), jnp.float32),
            pltpu.SemaphoreType.DMA,
            pltpu.SemaphoreType.DMA,
            pltpu.SemaphoreType.DMA,
            pltpu.SemaphoreType.DMA,
        ],
        compiler_params=pltpu.CompilerParams(
            needs_layout_passes=False, use_tc_tiling_on_sc=False),
    )(qwords, idx2d, scale_vec, bias_vec)


def kernel(input, qweight, scale, zero_point):
    nb, nf = input.shape
    qwords = _pack_words(qweight)
    idxT = jnp.swapaxes(input.astype(jnp.int32), 0, 1)
    idx2d = idxT.reshape(B_TOTAL // CHUNK, CHUNK)
    scale_f = scale.astype(jnp.float32)
    bias_f = -zero_point.astype(jnp.float32) * scale_f
    scale_vec = jnp.full((L,), scale_f, jnp.float32)
    bias_vec = jnp.full((L,), bias_f, jnp.float32)
    flat = _run(qwords, idx2d, scale_vec, bias_vec)
    return jnp.transpose(flat.reshape(nf, nb, D_BYTES), (1, 0, 2))


# speedup vs baseline: 3.4381x; 1.1121x over previous
import functools

import jax
import jax.numpy as jnp
from jax import lax
from jax.experimental import pallas as pl
from jax.experimental.pallas import tpu as pltpu
from jax.experimental.pallas import tpu_sc as plsc

NUM_EMB = 100000
D_BYTES = 128
B_TOTAL = 4096 * 26
NC, NS, L = 2, 16, 16
NW = NC * NS
ROWS_PER_W = B_TOTAL // NW
CHUNK = 128
CHUNKS_PER_W = ROWS_PER_W // CHUNK
TC_BLK = 20000


def _pack_body(in_ref, out_ref):
    out_ref[...] = pltpu.bitcast(in_ref[...], jnp.int32)


def _pack_words(qweight):
    return pl.pallas_call(
        _pack_body,
        grid=(NUM_EMB // TC_BLK,),
        in_specs=[pl.BlockSpec((TC_BLK, D_BYTES), lambda i: (i, 0))],
        out_specs=pl.BlockSpec((TC_BLK // 4, D_BYTES), lambda i: (i, 0)),
        out_shape=jax.ShapeDtypeStruct((NUM_EMB // 4, D_BYTES), jnp.int32),
    )(qweight)


def _body(qw_hbm, idx_hbm, scale_hbm, bias_hbm, out_hbm,
          idx_v, idxg0_v, idxg1_v, sh0_v, sh1_v,
          rows0_v, rows1_v, out0_v, out1_v, sv_v, bv_v,
          gsem0, gsem1, osem0, osem1):
    wid = lax.axis_index("s") * NC + lax.axis_index("c")

    pltpu.sync_copy(scale_hbm, sv_v)
    pltpu.sync_copy(bias_hbm, bv_v)
    pltpu.sync_copy(idx_hbm.at[pl.ds(wid * CHUNKS_PER_W, CHUNKS_PER_W)], idx_v)
    scale = sv_v[...]
    bias = bv_v[...]

    rows = (rows0_v, rows1_v)
    outs = (out0_v, out1_v)
    idxgs = (idxg0_v, idxg1_v)
    shs = (sh0_v, sh1_v)
    gsems = (gsem0, gsem1)
    osems = (osem0, osem1)

    def start_gather(c, b):
        for i in range(CHUNK // L):
            sl = pl.ds(L * i, L)
            v = idx_v[c, sl]
            idxgs[b][sl] = lax.shift_right_logical(v, 2)
            shs[b][sl] = lax.shift_left(1, 24 - lax.shift_left(v & 3, 3))
        return pltpu.async_copy(qw_hbm.at[idxgs[b]], rows[b], gsems[b])

    def wait_gather(b):
        pltpu.make_async_copy(qw_hbm.at[idxgs[b]], rows[b], gsems[b]).wait()

    def dequant(b):
        rows_b, out_b, sh_b = rows[b], outs[b], shs[b]

        @plsc.parallel_loop(0, CHUNK, 1, unroll=4)
        def _row(rr):
            mvec = plsc.load_gather(sh_b, [jnp.full((L,), rr, jnp.int32)])
            rowoff = rr * D_BYTES
            for w in range(D_BYTES // L):
                x = rows_b[rr, pl.ds(L * w, L)]
                t = lax.shift_right_arithmetic(x * mvec, 24)
                f = t.astype(jnp.float32) * scale + bias
                out_b[pl.ds(rowoff + L * w, L)] = f

    def out_slice(c):
        return out_hbm.at[pl.ds((wid * CHUNKS_PER_W + c) * CHUNK * D_BYTES,
                                CHUNK * D_BYTES)]

    def start_writeback(c, b):
        return pltpu.async_copy(outs[b], out_slice(c), osems[b])

    def wait_writeback(b):
        pltpu.make_async_copy(outs[b], out_slice(0), osems[b]).wait()

    start_gather(0, 0)

    def step(t, _):
        c0 = 2 * t
        start_gather(c0 + 1, 1)
        wait_gather(0)

        @pl.when(t > 0)
        def _():
            wait_writeback(0)
        dequant(0)
        start_writeback(c0, 0)

        @pl.when(t + 1 < CHUNKS_PER_W // 2)
        def _():
            start_gather(c0 + 2, 0)
        wait_gather(1)

        @pl.when(t > 0)
        def _():
            wait_writeback(1)
        dequant(1)
        start_writeback(c0 + 1, 1)
        return 0

    lax.fori_loop(0, CHUNKS_PER_W // 2, step, 0)
    wait_writeback(0)
    wait_writeback(1)


@jax.jit
def _run(qwords, idx2d, scale_vec, bias_vec):
    mesh = plsc.VectorSubcoreMesh(
        core_axis_name="c", subcore_axis_name="s",
        num_cores=NC, num_subcores=NS)
    return pl.kernel(
        _body,
        out_type=jax.ShapeDtypeStruct((B_TOTAL * D_BYTES,), jnp.float32),
        mesh=mesh,
        scratch_types=[
            pltpu.VMEM((CHUNKS_PER_W, CHUNK), jnp.int32),
            pltpu.VMEM((CHUNK,), jnp.int32),
            pltpu.VMEM((CHUNK,), jnp.int32),
            pltpu.VMEM((CHUNK,), jnp.int32),
            pltpu.VMEM((CHUNK,), jnp.int32),
            pltpu.VMEM((CHUNK, D_BYTES), jnp.int32),
            pltpu.VMEM((CHUNK, D_BYTES), jnp.int32),
            pltpu.VMEM((CHUNK * D_BYTES,), jnp.float32),
            pltpu.VMEM((CHUNK * D_BYTES,), jnp.float32),
            pltpu.VMEM((L,), jnp.float32),
            pltpu.VMEM((L,), jnp.float32),
            pltpu.SemaphoreType.DMA,
            pltpu.SemaphoreType.DMA,
            pltpu.SemaphoreType.DMA,
            pltpu.SemaphoreType.DMA,
        ],
        compiler_params=pltpu.CompilerParams(
            needs_layout_passes=False, use_tc_tiling_on_sc=False),
    )(qwords, idx2d, scale_vec, bias_vec)


def kernel(input, qweight, scale, zero_point):
    nb, nf = input.shape
    qwords = _pack_words(qweight)
    idxT = jnp.swapaxes(input.astype(jnp.int32), 0, 1)
    idx2d = idxT.reshape(B_TOTAL // CHUNK, CHUNK)
    scale_f = scale.astype(jnp.float32)
    bias_f = -zero_point.astype(jnp.float32) * scale_f
    scale_vec = jnp.full((L,), scale_f, jnp.float32)
    bias_vec = jnp.full((L,), bias_f, jnp.float32)
    flat = _run(qwords, idx2d, scale_vec, bias_vec)
    return jnp.transpose(flat.reshape(nf, nb, D_BYTES), (1, 0, 2))


# speedup vs baseline: 3.4867x; 1.0141x over previous
import functools

import jax
import jax.numpy as jnp
from jax import lax
from jax.experimental import pallas as pl
from jax.experimental.pallas import tpu as pltpu
from jax.experimental.pallas import tpu_sc as plsc

NUM_EMB = 100000
D_BYTES = 128
B_TOTAL = 4096 * 26
NC, NS, L = 2, 16, 16
NW = NC * NS
ROWS_PER_W = B_TOTAL // NW
CHUNK = 128
CHUNKS_PER_W = ROWS_PER_W // CHUNK
TC_BLK = 20000


def _pack_body(in_ref, out_ref):
    out_ref[...] = pltpu.bitcast(in_ref[...], jnp.int32)


def _pack_words(qweight):
    return pl.pallas_call(
        _pack_body,
        grid=(NUM_EMB // TC_BLK,),
        in_specs=[pl.BlockSpec((TC_BLK, D_BYTES), lambda i: (i, 0))],
        out_specs=pl.BlockSpec((TC_BLK // 4, D_BYTES), lambda i: (i, 0)),
        out_shape=jax.ShapeDtypeStruct((NUM_EMB // 4, D_BYTES), jnp.int32),
    )(qweight)


def _body(qw_hbm, idx_hbm, scale_hbm, bias_hbm, out_hbm,
          idx_v,
          idxg00_v, idxg01_v, idxg10_v, idxg11_v,
          sh00_v, sh01_v, sh10_v, sh11_v,
          rows00_v, rows01_v, rows10_v, rows11_v,
          out0_v, out1_v, sv_v, bv_v,
          gsem00, gsem01, gsem10, gsem11, osem0, osem1):
    wid = lax.axis_index("s") * NC + lax.axis_index("c")

    pltpu.sync_copy(scale_hbm, sv_v)
    pltpu.sync_copy(bias_hbm, bv_v)
    pltpu.sync_copy(idx_hbm.at[pl.ds(wid * CHUNKS_PER_W, CHUNKS_PER_W)], idx_v)
    scale = sv_v[...]
    bias = bv_v[...]

    rows = ((rows00_v, rows01_v), (rows10_v, rows11_v))
    idxgs = ((idxg00_v, idxg01_v), (idxg10_v, idxg11_v))
    shs = ((sh00_v, sh01_v), (sh10_v, sh11_v))
    gsems = ((gsem00, gsem01), (gsem10, gsem11))
    outs = (out0_v, out1_v)
    osems = (osem0, osem1)

    def start_gather(c, p, j):
        for i in range(CHUNK // L):
            sl = pl.ds(L * i, L)
            v = idx_v[c, sl]
            idxgs[p][j][sl] = lax.shift_right_logical(v, 2)
            shs[p][j][sl] = lax.shift_left(1, 24 - lax.shift_left(v & 3, 3))
        pltpu.async_copy(qw_hbm.at[idxgs[p][j]], rows[p][j], gsems[p][j])

    def wait_gather(p, j):
        pltpu.make_async_copy(
            qw_hbm.at[idxgs[p][j]], rows[p][j], gsems[p][j]).wait()

    def dequant(p, j):
        rows_b, out_b, sh_b = rows[p][j], outs[j], shs[p][j]

        @plsc.parallel_loop(0, CHUNK, 1, unroll=4)
        def _row(rr):
            mvec = plsc.load_gather(sh_b, [jnp.full((L,), rr, jnp.int32)])
            rowoff = rr * D_BYTES
            for w in range(D_BYTES // L):
                x = rows_b[rr, pl.ds(L * w, L)]
                t = lax.shift_right_arithmetic(x * mvec, 24)
                f = t.astype(jnp.float32) * scale + bias
                out_b[pl.ds(rowoff + L * w, L)] = f

    def out_slice(c):
        return out_hbm.at[pl.ds((wid * CHUNKS_PER_W + c) * CHUNK * D_BYTES,
                                CHUNK * D_BYTES)]

    def start_writeback(c, j):
        pltpu.async_copy(outs[j], out_slice(c), osems[j])

    def wait_writeback(j):
        pltpu.make_async_copy(outs[j], out_slice(0), osems[j]).wait()

    start_gather(0, 0, 0)
    start_gather(1, 0, 1)
    start_gather(2, 1, 0)
    start_gather(3, 1, 1)

    def process_pair(c0, p, first):
        for j in range(2):
            wait_gather(p, j)
            if first:
                @pl.when(c0 > 0)
                def _():
                    wait_writeback(j)
            else:
                wait_writeback(j)
            dequant(p, j)
            start_writeback(c0 + j, j)

    def step(t, _):
        c0 = 4 * t
        process_pair(c0, 0, True)

        @pl.when(c0 + 4 < CHUNKS_PER_W)
        def _():
            start_gather(c0 + 4, 0, 0)
            start_gather(c0 + 5, 0, 1)
        process_pair(c0 + 2, 1, False)

        @pl.when(c0 + 6 < CHUNKS_PER_W)
        def _():
            start_gather(c0 + 6, 1, 0)
            start_gather(c0 + 7, 1, 1)
        return 0

    lax.fori_loop(0, CHUNKS_PER_W // 4, step, 0)
    process_pair(CHUNKS_PER_W - 2, 0, False)
    wait_writeback(0)
    wait_writeback(1)


@jax.jit
def _run(qwords, idx2d, scale_vec, bias_vec):
    mesh = plsc.VectorSubcoreMesh(
        core_axis_name="c", subcore_axis_name="s",
        num_cores=NC, num_subcores=NS)
    return pl.kernel(
        _body,
        out_type=jax.ShapeDtypeStruct((B_TOTAL * D_BYTES,), jnp.float32),
        mesh=mesh,
        scratch_types=(
            [pltpu.VMEM((CHUNKS_PER_W, CHUNK), jnp.int32)]
            + [pltpu.VMEM((CHUNK,), jnp.int32)] * 4
            + [pltpu.VMEM((CHUNK,), jnp.int32)] * 4
            + [pltpu.VMEM((CHUNK, D_BYTES), jnp.int32)] * 4
            + [pltpu.VMEM((CHUNK * D_BYTES,), jnp.float32)] * 2
            + [pltpu.VMEM((L,), jnp.float32)] * 2
            + [pltpu.SemaphoreType.DMA] * 6
        ),
        compiler_params=pltpu.CompilerParams(
            needs_layout_passes=False, use_tc_tiling_on_sc=False),
    )(qwords, idx2d, scale_vec, bias_vec)


def kernel(input, qweight, scale, zero_point):
    nb, nf = input.shape
    qwords = _pack_words(qweight)
    idxT = jnp.swapaxes(input.astype(jnp.int32), 0, 1)
    idx2d = idxT.reshape(B_TOTAL // CHUNK, CHUNK)
    scale_f = scale.astype(jnp.float32)
    bias_f = -zero_point.astype(jnp.float32) * scale_f
    scale_vec = jnp.full((L,), scale_f, jnp.float32)
    bias_vec = jnp.full((L,), bias_f, jnp.float32)
    flat = _run(qwords, idx2d, scale_vec, bias_vec)
    return jnp.transpose(flat.reshape(nf, nb, D_BYTES), (1, 0, 2))
